# baseline (speedup 1.0000x reference)
import functools

import jax
import jax.numpy as jnp
from jax import lax
from jax.experimental import pallas as pl
from jax.experimental.pallas import tpu as pltpu
from jax.experimental.pallas import tpu_sc as plsc

_S = 8192
_D = 2048
_B = 4
_NC = 2
_NS = 16
_NW = _NC * _NS
_ROWS_PER_W = _S // _NW
_LENS = [56, 56, 56, 56, 32]
assert sum(_LENS) == _ROWS_PER_W
_OFFS = [sum(_LENS[:i]) for i in range(len(_LENS))]
_NCHUNK = len(_LENS)

_mesh = plsc.VectorSubcoreMesh(core_axis_name="c", subcore_axis_name="s")


@functools.partial(
    pl.kernel,
    mesh=_mesh,
    out_type=jax.ShapeDtypeStruct((_B * _S, _D), jnp.float32),
    scratch_types=[
        pltpu.VMEM((56, _D), jnp.float32),
        pltpu.SemaphoreType.DMA,
        pltpu.SemaphoreType.DMA,
    ],
)
def _bcast_rows(table_hbm, out_hbm, buf, rsem, wsem):
    wid = lax.axis_index("s") * _NC + lax.axis_index("c")
    base = wid * _ROWS_PER_W

    for i in range(_NCHUNK):
        pltpu.async_copy(
            table_hbm.at[pl.ds(base + _OFFS[i], _LENS[i])],
            buf.at[pl.ds(0, _LENS[i])],
            rsem,
        ).wait()
        whs = [
            pltpu.async_copy(
                buf.at[pl.ds(0, _LENS[i])],
                out_hbm.at[pl.ds(b * _S + base + _OFFS[i], _LENS[i])],
                wsem,
            )
            for b in range(_B)
        ]
        for c in whs:
            c.wait()


def kernel(x, table):
    del x
    out = _bcast_rows(table)
    return out.reshape(_B, _S, _D)
